# baseline (speedup 1.0000x reference)
import jax
import jax.numpy as jnp
from jax.experimental import pallas as pl
from jax.experimental.pallas import tpu as pltpu

_NMS_THRESHOLD = 8.0
_PRE_NMS_TOPK = 5000
_MAX_OUTPUT = 2560
_MIN_SCORE = 0.2

_BLK = 256
_NBLK = 20
_NPAD = _BLK * _NBLK


def _nms_kernel(s_row_ref, x_row_ref, y_row_ref, s_col_ref, x_col_ref,
                y_col_ref, scores_out_ref, locs_out_ref, keep_ref):
    d2_thr = _NMS_THRESHOLD * _NMS_THRESHOLD

    ii = jax.lax.broadcasted_iota(jnp.int32, (_BLK, _BLK), 0)
    jj = jax.lax.broadcasted_iota(jnp.int32, (_BLK, _BLK), 1)
    strict_upper = jnp.where(jj > ii, 1.0, 0.0)

    keep_ref[...] = jnp.zeros((_NBLK, _BLK), jnp.float32)

    for b in range(_NBLK):
        o = b * _BLK
        xb_r = x_row_ref[b:b + 1, :]
        yb_r = y_row_ref[b:b + 1, :]
        sb_r = s_row_ref[b:b + 1, :]
        xb_c = x_col_ref[o:o + _BLK, :]
        yb_c = y_col_ref[o:o + _BLK, :]
        valid = jnp.where(sb_r >= _MIN_SCORE, 1.0, 0.0)

        def cross_body(p, acc):
            q = p * _BLK
            xp_c = x_col_ref[pl.ds(q, _BLK), :]
            yp_c = y_col_ref[pl.ds(q, _BLK), :]
            kp_r = keep_ref[pl.ds(p, 1), :]
            dx = xp_c - xb_r
            dy = yp_c - yb_r
            hit = jnp.where(dx * dx + dy * dy < d2_thr, 1.0, 0.0)
            return acc + jax.lax.dot(kp_r, hit,
                                     preferred_element_type=jnp.float32)

        cross = jnp.zeros((1, _BLK), jnp.float32)
        if b > 0:
            cross = jax.lax.fori_loop(0, b, cross_body, cross)

        dxb = xb_c - xb_r
        dyb = yb_c - yb_r
        sup = jnp.where(dxb * dxb + dyb * dyb < d2_thr, 1.0, 0.0) * strict_upper

        undec = valid * jnp.where(cross > 0.5, 0.0, 1.0)
        kept = jnp.zeros((1, _BLK), jnp.float32)

        def rounds_cond(state):
            u, _ = state
            return jnp.sum(u) > 0.0

        def rounds_body(state):
            u, k = state
            uv = jax.lax.dot(u, sup, preferred_element_type=jnp.float32)
            newly = u * jnp.where(uv > 0.5, 0.0, 1.0)
            killed = jax.lax.dot(newly, sup, preferred_element_type=jnp.float32)
            k2 = k + newly
            u2 = u * (1.0 - newly) * jnp.where(killed > 0.5, 0.0, 1.0)
            return u2, k2

        _, keep_blk = jax.lax.while_loop(rounds_cond, rounds_body,
                                         (undec, kept))
        keep_ref[b:b + 1, :] = keep_blk

    keep_mat = keep_ref[...]
    incl = jnp.where(jj >= ii, 1.0, 0.0)
    c_in = jax.lax.dot(keep_mat, incl, preferred_element_type=jnp.float32)
    tot = jax.lax.dot(keep_mat, jnp.ones((_BLK, 1), jnp.float32),
                      preferred_element_type=jnp.float32)
    bi = jax.lax.broadcasted_iota(jnp.int32, (_NBLK, _NBLK), 0)
    bj = jax.lax.broadcasted_iota(jnp.int32, (_NBLK, _NBLK), 1)
    blower = jnp.where(bj < bi, 1.0, 0.0)
    off = jax.lax.dot(blower, tot, preferred_element_type=jnp.float32)
    ranks = c_in + off

    slots = jax.lax.broadcasted_iota(jnp.int32, (_MAX_OUTPUT, 1), 0).astype(jnp.float32)
    acc_s = jnp.zeros((_MAX_OUTPUT, 1), jnp.float32)
    acc_x = jnp.zeros((_MAX_OUTPUT, 1), jnp.float32)
    acc_y = jnp.zeros((_MAX_OUTPUT, 1), jnp.float32)
    cnt = jnp.zeros((_MAX_OUTPUT, 1), jnp.float32)
    for b in range(_NBLK):
        rb = ranks[b:b + 1, :]
        kb = keep_mat[b:b + 1, :]
        onehot = jnp.where((rb - 1.0 == slots) & (kb > 0.5), 1.0, 0.0)
        acc_s = acc_s + jnp.sum(onehot * s_row_ref[b:b + 1, :],
                                axis=1, keepdims=True)
        acc_x = acc_x + jnp.sum(onehot * x_row_ref[b:b + 1, :],
                                axis=1, keepdims=True)
        acc_y = acc_y + jnp.sum(onehot * y_row_ref[b:b + 1, :],
                                axis=1, keepdims=True)
        cnt = cnt + jnp.sum(onehot, axis=1, keepdims=True)
    scores_out_ref[...] = jnp.where(cnt > 0.5, acc_s, -1.0)
    locs_out_ref[...] = jnp.where(cnt > 0.5,
                                  jnp.concatenate([acc_x, acc_y], axis=1),
                                  -1.0)


def _run_nms(scores_sorted, locs_sorted):
    pad = _NPAD - scores_sorted.shape[0]
    s_p = jnp.concatenate([scores_sorted, jnp.full((pad,), -1.0, jnp.float32)])
    x_p = jnp.concatenate([locs_sorted[:, 0], jnp.full((pad,), -1.0, jnp.float32)])
    y_p = jnp.concatenate([locs_sorted[:, 1], jnp.full((pad,), -1.0, jnp.float32)])
    out_scores, out_locs = pl.pallas_call(
        _nms_kernel,
        out_shape=[
            jax.ShapeDtypeStruct((_MAX_OUTPUT, 1), jnp.float32),
            jax.ShapeDtypeStruct((_MAX_OUTPUT, 2), jnp.float32),
        ],
        scratch_shapes=[
            pltpu.VMEM((_NBLK, _BLK), jnp.float32),
        ],
    )(
        s_p.reshape(_NBLK, _BLK),
        x_p.reshape(_NBLK, _BLK),
        y_p.reshape(_NBLK, _BLK),
        s_p.reshape(_NPAD, 1),
        x_p.reshape(_NPAD, 1),
        y_p.reshape(_NPAD, 1),
    )
    return out_scores.reshape(-1), out_locs


def kernel(scores_0, regressions_0, scores_1, regressions_1):
    scores = {"0": scores_0, "1": scores_1}
    regressions = {"0": regressions_0, "1": regressions_1}
    scores_list = []
    loc_list = []
    for k in ("0", "1"):
        s = scores[k].squeeze(-1)
        r = regressions[k]
        height, width = s.shape
        loc = jnp.mgrid[:height, :width] + 0.5
        loc = loc.transpose(1, 2, 0) + r
        is_valid = ((loc > 0.0).all(axis=-1)
                    & (loc[:, :, 0] < height) & (loc[:, :, 1] < width))
        s = jnp.where(is_valid, s, 0)
        loc = loc * (2 ** int(k))
        loc_list.append(loc.reshape(-1, 2))
        scores_list.append(s.reshape(-1))
    scores_flat = jnp.concatenate(scores_list, axis=0)
    locations = jnp.concatenate(loc_list, axis=0)
    top_scores, selections = jax.lax.top_k(scores_flat, _PRE_NMS_TOPK)
    top_locs = locations[selections]
    return _run_nms(top_scores, top_locs)


# speedup vs baseline: 1.7888x; 1.7888x over previous
import jax
import jax.numpy as jnp
from jax.experimental import pallas as pl
from jax.experimental.pallas import tpu as pltpu

_NMS_THRESHOLD = 8.0
_PRE_NMS_TOPK = 5000
_MAX_OUTPUT = 2560
_MIN_SCORE = 0.2

_BLK = 256
_NBLK = 20
_NPAD = _BLK * _NBLK


def _nms_kernel(s_row_ref, x_row_ref, y_row_ref, s_col_ref, x_col_ref,
                y_col_ref, scores_out_ref, locs_out_ref, keep_ref):
    d2_thr = _NMS_THRESHOLD * _NMS_THRESHOLD

    ii = jax.lax.broadcasted_iota(jnp.int32, (_BLK, _BLK), 0)
    jj = jax.lax.broadcasted_iota(jnp.int32, (_BLK, _BLK), 1)
    strict_upper = jnp.where(jj > ii, 1.0, 0.0)

    keep_ref[...] = jnp.zeros((_NBLK, _BLK), jnp.float32)

    for b in range(_NBLK):
        o = b * _BLK
        xb_r = x_row_ref[b:b + 1, :]
        yb_r = y_row_ref[b:b + 1, :]
        sb_r = s_row_ref[b:b + 1, :]
        xb_c = x_col_ref[o:o + _BLK, :]
        yb_c = y_col_ref[o:o + _BLK, :]
        valid = jnp.where(sb_r >= _MIN_SCORE, 1.0, 0.0)

        def cross_body(p, acc):
            q = p * _BLK
            xp_c = x_col_ref[pl.ds(q, _BLK), :]
            yp_c = y_col_ref[pl.ds(q, _BLK), :]
            kp_r = keep_ref[pl.ds(p, 1), :]
            dx = xp_c - xb_r
            dy = yp_c - yb_r
            hit = jnp.where(dx * dx + dy * dy < d2_thr, 1.0, 0.0)
            return acc + jax.lax.dot(kp_r, hit,
                                     preferred_element_type=jnp.float32)

        cross = jnp.zeros((1, _BLK), jnp.float32)
        if b > 0:
            cross = jax.lax.fori_loop(0, b, cross_body, cross)

        dxb = xb_c - xb_r
        dyb = yb_c - yb_r
        sup = jnp.where(dxb * dxb + dyb * dyb < d2_thr, 1.0, 0.0) * strict_upper

        undec = valid * jnp.where(cross > 0.5, 0.0, 1.0)
        kept = jnp.zeros((1, _BLK), jnp.float32)

        def rounds_cond(state):
            u, _ = state
            return jnp.sum(u) > 0.0

        def rounds_body(state):
            u, k = state
            uv = jax.lax.dot(u, sup, preferred_element_type=jnp.float32)
            newly = u * jnp.where(uv > 0.5, 0.0, 1.0)
            killed = jax.lax.dot(newly, sup, preferred_element_type=jnp.float32)
            k2 = k + newly
            u2 = u * (1.0 - newly) * jnp.where(killed > 0.5, 0.0, 1.0)
            return u2, k2

        _, keep_blk = jax.lax.while_loop(rounds_cond, rounds_body,
                                         (undec, kept))
        keep_ref[b:b + 1, :] = keep_blk

    keep_mat = keep_ref[...]
    incl = jnp.where(jj >= ii, 1.0, 0.0)
    c_in = jax.lax.dot(keep_mat, incl, preferred_element_type=jnp.float32)
    tot = jax.lax.dot(keep_mat, jnp.ones((_BLK, 1), jnp.float32),
                      preferred_element_type=jnp.float32)
    bi = jax.lax.broadcasted_iota(jnp.int32, (_NBLK, _NBLK), 0)
    bj = jax.lax.broadcasted_iota(jnp.int32, (_NBLK, _NBLK), 1)
    blower = jnp.where(bj < bi, 1.0, 0.0)
    off = jax.lax.dot(blower, tot, preferred_element_type=jnp.float32)
    ranks = c_in + off

    slots = jax.lax.broadcasted_iota(jnp.int32, (_MAX_OUTPUT, 1), 0).astype(jnp.float32)
    acc_s = jnp.zeros((_MAX_OUTPUT, 1), jnp.float32)
    acc_x = jnp.zeros((_MAX_OUTPUT, 1), jnp.float32)
    acc_y = jnp.zeros((_MAX_OUTPUT, 1), jnp.float32)
    cnt = jnp.zeros((_MAX_OUTPUT, 1), jnp.float32)
    for b in range(_NBLK):
        rb = ranks[b:b + 1, :]
        kb = keep_mat[b:b + 1, :]
        onehot = jnp.where((rb - 1.0 == slots) & (kb > 0.5), 1.0, 0.0)
        acc_s = acc_s + jnp.sum(onehot * s_row_ref[b:b + 1, :],
                                axis=1, keepdims=True)
        acc_x = acc_x + jnp.sum(onehot * x_row_ref[b:b + 1, :],
                                axis=1, keepdims=True)
        acc_y = acc_y + jnp.sum(onehot * y_row_ref[b:b + 1, :],
                                axis=1, keepdims=True)
        cnt = cnt + jnp.sum(onehot, axis=1, keepdims=True)
    scores_out_ref[...] = jnp.where(cnt > 0.5, acc_s, -1.0)
    locs_out_ref[...] = jnp.where(cnt > 0.5,
                                  jnp.concatenate([acc_x, acc_y], axis=1),
                                  -1.0)


def _run_nms(scores_sorted, locs_sorted):
    pad = _NPAD - scores_sorted.shape[0]
    s_p = jnp.concatenate([scores_sorted, jnp.full((pad,), -1.0, jnp.float32)])
    x_p = jnp.concatenate([locs_sorted[:, 0], jnp.full((pad,), -1.0, jnp.float32)])
    y_p = jnp.concatenate([locs_sorted[:, 1], jnp.full((pad,), -1.0, jnp.float32)])
    out_scores, out_locs = pl.pallas_call(
        _nms_kernel,
        out_shape=[
            jax.ShapeDtypeStruct((_MAX_OUTPUT, 1), jnp.float32),
            jax.ShapeDtypeStruct((_MAX_OUTPUT, 2), jnp.float32),
        ],
        scratch_shapes=[
            pltpu.VMEM((_NBLK, _BLK), jnp.float32),
        ],
    )(
        s_p.reshape(_NBLK, _BLK),
        x_p.reshape(_NBLK, _BLK),
        y_p.reshape(_NBLK, _BLK),
        s_p.reshape(_NPAD, 1),
        x_p.reshape(_NPAD, 1),
        y_p.reshape(_NPAD, 1),
    )
    return out_scores.reshape(-1), out_locs


def kernel(scores_0, regressions_0, scores_1, regressions_1):
    scores = {"0": scores_0, "1": scores_1}
    regressions = {"0": regressions_0, "1": regressions_1}
    scores_list = []
    loc_list = []
    for k in ("0", "1"):
        s = scores[k].squeeze(-1)
        r = regressions[k]
        height, width = s.shape
        loc = jnp.mgrid[:height, :width] + 0.5
        loc = loc.transpose(1, 2, 0) + r
        is_valid = ((loc > 0.0).all(axis=-1)
                    & (loc[:, :, 0] < height) & (loc[:, :, 1] < width))
        s = jnp.where(is_valid, s, 0)
        loc = loc * (2 ** int(k))
        loc_list.append(loc.reshape(-1, 2))
        scores_list.append(s.reshape(-1))
    scores_flat = jnp.concatenate(scores_list, axis=0)
    locations = jnp.concatenate(loc_list, axis=0)

    _FILTER = 0.9
    _CAP = 36864
    mask = scores_flat > _FILTER
    cnt = jnp.sum(mask.astype(jnp.int32))

    def _fast(_):
        idx = jnp.where(mask, size=_CAP, fill_value=0)[0]
        sg = jnp.where(jnp.arange(_CAP) < cnt, scores_flat[idx], -1.0)
        ts, sl = jax.lax.top_k(sg, _PRE_NMS_TOPK)
        return ts, idx[sl]

    def _slow(_):
        ts, sl = jax.lax.top_k(scores_flat, _PRE_NMS_TOPK)
        return ts, sl

    top_scores, selections = jax.lax.cond(
        (cnt >= _PRE_NMS_TOPK) & (cnt <= _CAP), _fast, _slow, None)
    top_locs = locations[selections]
    return _run_nms(top_scores, top_locs)
